# baseline (speedup 1.0000x reference)
import functools

import jax
import jax.numpy as jnp
from jax import lax
from jax.experimental import pallas as pl
from jax.experimental.pallas import tpu as pltpu
from jax.experimental.pallas import tpu_sc as plsc

NUM_EXPERTS = 64
TOP_K = 8
LANES = 16
NUM_CORES = 2
NUM_SUBCORES = 16
NUM_WORKERS = NUM_CORES * NUM_SUBCORES
NUM_CHUNKS = 4
UNROLL = 2


def _tec_kernel(T, conf_hbm, wealth_hbm, eidx_hbm, wgt_hbm, pay_hbm,
                conf_v0, conf_v1, w_v, eidx_v, wgt_v, pay_v, sem0, sem1):
  wid = lax.axis_index("s") * NUM_CORES + lax.axis_index("c")
  S = conf_hbm.shape[1]
  wpb = S // T
  b = wid // wpb
  s0 = (wid % wpb) * T

  pltpu.sync_copy(wealth_hbm, w_v)

  w0 = w_v[pl.ds(0, LANES)]
  w1 = w_v[pl.ds(16, LANES)]
  w2 = w_v[pl.ds(32, LANES)]
  w3 = w_v[pl.ds(48, LANES)]

  iota = lax.iota(jnp.int32, LANES)
  idx0 = iota
  idx1 = iota + 16
  idx2 = iota + 32
  idx3 = iota + 48
  lane_lt8 = iota < TOP_K
  k128 = (iota & 7) * 128

  def merge_top16(ak, av, bk, bv):
    rbk = lax.rev(bk, (0,))
    rbv = lax.rev(bv, (0,))
    c = ak >= rbk
    return jnp.where(c, ak, rbk), jnp.where(c, av, rbv)

  def body(t, coff, conf_v):
    b0 = conf_v[t, pl.ds(0, LANES)] * w0
    b1 = conf_v[t, pl.ds(16, LANES)] * w1
    b2 = conf_v[t, pl.ds(32, LANES)] * w2
    b3 = conf_v[t, pl.ds(48, LANES)] * w3

    s0k, s0v = plsc.sort_key_val(b0, idx0, descending=True)
    s1k, s1v = plsc.sort_key_val(b1, idx1, descending=True)
    s2k, s2v = plsc.sort_key_val(b2, idx2, descending=True)
    s3k, s3v = plsc.sort_key_val(b3, idx3, descending=True)

    h01k, h01v = merge_top16(s0k, s0v, s1k, s1v)
    h23k, h23v = merge_top16(s2k, s2v, s3k, s3v)
    m01k, m01v = plsc.sort_key_val(h01k, h01v, descending=True)
    m23k, m23v = plsc.sort_key_val(h23k, h23v, descending=True)
    hk, hv = merge_top16(m01k, m01v, m23k, m23v)
    fk, fv = plsc.sort_key_val(hk, hv, descending=True)

    pay = jnp.max(jnp.where(lane_lt8, -1.0, fk))
    e = jnp.where(lane_lt8, jnp.exp(fk), 0.0)
    wgt = e / jnp.sum(e)

    tl = coff + t
    oidx = [k128 + ((tl >> 7) * (TOP_K * 128) + (tl & 127))]
    plsc.store_scatter(eidx_v, oidx, fv, mask=lane_lt8)
    plsc.store_scatter(wgt_v, oidx, wgt, mask=lane_lt8)
    plsc.store_scatter(pay_v, oidx, jnp.full_like(fk, pay), mask=lane_lt8)

  C = T // NUM_CHUNKS
  bufs = (conf_v0, conf_v1)
  sems = (sem0, sem1)

  def copy(ch):
    return pltpu.make_async_copy(
        conf_hbm.at[b, pl.ds(s0 + ch * C, C)], bufs[ch % 2], sems[ch % 2])

  copy(0).start()
  for ch in range(NUM_CHUNKS):
    if ch + 1 < NUM_CHUNKS:
      copy(ch + 1).start()
    copy(ch).wait()
    plsc.parallel_loop(0, C, unroll=UNROLL)(
        lambda t, coff=ch * C, cv=bufs[ch % 2]: body(t, coff, cv))

  off = (b * S + s0) * TOP_K
  sz = T * TOP_K
  pltpu.sync_copy(eidx_v, eidx_hbm.at[pl.ds(off, sz)])
  pltpu.sync_copy(wgt_v, wgt_hbm.at[pl.ds(off, sz)])
  pltpu.sync_copy(pay_v, pay_hbm.at[pl.ds(off, sz)])


@jax.jit
def kernel(confidences, wealth):
  B, S, E = confidences.shape
  N = B * S
  T = N // NUM_WORKERS
  NT = T // 128

  mesh = plsc.VectorSubcoreMesh(
      core_axis_name="c", subcore_axis_name="s",
      num_cores=NUM_CORES, num_subcores=NUM_SUBCORES)

  eidx, wgt, pay = pl.kernel(
      functools.partial(_tec_kernel, T),
      out_type=(
          jax.ShapeDtypeStruct((N * TOP_K,), jnp.int32),
          jax.ShapeDtypeStruct((N * TOP_K,), jnp.float32),
          jax.ShapeDtypeStruct((N * TOP_K,), jnp.float32),
      ),
      mesh=mesh,
      compiler_params=pltpu.CompilerParams(needs_layout_passes=False),
      scratch_types=[
          pltpu.VMEM((T // NUM_CHUNKS, E), jnp.float32),
          pltpu.VMEM((T // NUM_CHUNKS, E), jnp.float32),
          pltpu.VMEM((E,), jnp.float32),
          pltpu.VMEM((T * TOP_K,), jnp.int32),
          pltpu.VMEM((T * TOP_K,), jnp.float32),
          pltpu.VMEM((T * TOP_K,), jnp.float32),
          pltpu.SemaphoreType.DMA,
          pltpu.SemaphoreType.DMA,
      ],
  )(confidences, wealth)

  def detile(x):
    x = x.reshape(B, S // 128, TOP_K, 128)
    return x.transpose(0, 1, 3, 2).reshape(B, S, TOP_K)

  return (detile(eidx), detile(wgt), detile(pay))


# speedup vs baseline: 1.0236x; 1.0236x over previous
import functools

import jax
import jax.numpy as jnp
from jax import lax
from jax.experimental import pallas as pl
from jax.experimental.pallas import tpu as pltpu
from jax.experimental.pallas import tpu_sc as plsc

NUM_EXPERTS = 64
TOP_K = 8
LANES = 16
NUM_CORES = 2
NUM_SUBCORES = 16
NUM_WORKERS = NUM_CORES * NUM_SUBCORES
NUM_CHUNKS = 4
UNROLL = 3


def _tec_kernel(T, conf_hbm, wealth_hbm, eidx_hbm, wgt_hbm, pay_hbm,
                conf_v0, conf_v1, w_v, eidx_v, wgt_v, pay_v, sem0, sem1):
  wid = lax.axis_index("s") * NUM_CORES + lax.axis_index("c")
  S = conf_hbm.shape[1]
  wpb = S // T
  b = wid // wpb
  s0 = (wid % wpb) * T

  pltpu.sync_copy(wealth_hbm, w_v)

  w0 = w_v[pl.ds(0, LANES)]
  w1 = w_v[pl.ds(16, LANES)]
  w2 = w_v[pl.ds(32, LANES)]
  w3 = w_v[pl.ds(48, LANES)]

  iota = lax.iota(jnp.int32, LANES)
  idx0 = iota
  idx1 = iota + 16
  idx2 = iota + 32
  idx3 = iota + 48
  lane_lt8 = iota < TOP_K
  k128 = (iota & 7) * 128

  def merge_top16(ak, av, bk, bv):
    rbk = lax.rev(bk, (0,))
    rbv = lax.rev(bv, (0,))
    c = ak >= rbk
    return jnp.where(c, ak, rbk), jnp.where(c, av, rbv)

  def body(t, coff, conf_v):
    b0 = conf_v[t, pl.ds(0, LANES)] * w0
    b1 = conf_v[t, pl.ds(16, LANES)] * w1
    b2 = conf_v[t, pl.ds(32, LANES)] * w2
    b3 = conf_v[t, pl.ds(48, LANES)] * w3

    s0k, s0v = plsc.sort_key_val(b0, idx0, descending=True)
    s1k, s1v = plsc.sort_key_val(b1, idx1, descending=True)
    s2k, s2v = plsc.sort_key_val(b2, idx2, descending=True)
    s3k, s3v = plsc.sort_key_val(b3, idx3, descending=True)

    h01k, h01v = merge_top16(s0k, s0v, s1k, s1v)
    h23k, h23v = merge_top16(s2k, s2v, s3k, s3v)
    m01k, m01v = plsc.sort_key_val(h01k, h01v, descending=True)
    m23k, m23v = plsc.sort_key_val(h23k, h23v, descending=True)
    hk, hv = merge_top16(m01k, m01v, m23k, m23v)
    fk, fv = plsc.sort_key_val(hk, hv, descending=True)

    pay = jnp.max(jnp.where(lane_lt8, -1.0, fk))
    e = jnp.where(lane_lt8, jnp.exp(fk), 0.0)
    wgt = e / jnp.sum(e)

    tl = coff + t
    oidx = [k128 + ((tl >> 7) * (TOP_K * 128) + (tl & 127))]
    plsc.store_scatter(eidx_v, oidx, fv, mask=lane_lt8)
    plsc.store_scatter(wgt_v, oidx, wgt, mask=lane_lt8)
    plsc.store_scatter(pay_v, oidx, jnp.full_like(fk, pay), mask=lane_lt8)

  C = T // NUM_CHUNKS
  bufs = (conf_v0, conf_v1)
  sems = (sem0, sem1)

  def copy(ch):
    return pltpu.make_async_copy(
        conf_hbm.at[b, pl.ds(s0 + ch * C, C)], bufs[ch % 2], sems[ch % 2])

  copy(0).start()
  for ch in range(NUM_CHUNKS):
    if ch + 1 < NUM_CHUNKS:
      copy(ch + 1).start()
    copy(ch).wait()
    plsc.parallel_loop(0, C, unroll=UNROLL)(
        lambda t, coff=ch * C, cv=bufs[ch % 2]: body(t, coff, cv))

  off = (b * S + s0) * TOP_K
  sz = T * TOP_K
  pltpu.sync_copy(eidx_v, eidx_hbm.at[pl.ds(off, sz)])
  pltpu.sync_copy(wgt_v, wgt_hbm.at[pl.ds(off, sz)])
  pltpu.sync_copy(pay_v, pay_hbm.at[pl.ds(off, sz)])


@jax.jit
def kernel(confidences, wealth):
  B, S, E = confidences.shape
  N = B * S
  T = N // NUM_WORKERS
  NT = T // 128

  mesh = plsc.VectorSubcoreMesh(
      core_axis_name="c", subcore_axis_name="s",
      num_cores=NUM_CORES, num_subcores=NUM_SUBCORES)

  eidx, wgt, pay = pl.kernel(
      functools.partial(_tec_kernel, T),
      out_type=(
          jax.ShapeDtypeStruct((N * TOP_K,), jnp.int32),
          jax.ShapeDtypeStruct((N * TOP_K,), jnp.float32),
          jax.ShapeDtypeStruct((N * TOP_K,), jnp.float32),
      ),
      mesh=mesh,
      compiler_params=pltpu.CompilerParams(needs_layout_passes=False),
      scratch_types=[
          pltpu.VMEM((T // NUM_CHUNKS, E), jnp.float32),
          pltpu.VMEM((T // NUM_CHUNKS, E), jnp.float32),
          pltpu.VMEM((E,), jnp.float32),
          pltpu.VMEM((T * TOP_K,), jnp.int32),
          pltpu.VMEM((T * TOP_K,), jnp.float32),
          pltpu.VMEM((T * TOP_K,), jnp.float32),
          pltpu.SemaphoreType.DMA,
          pltpu.SemaphoreType.DMA,
      ],
  )(confidences, wealth)

  def detile(x):
    x = x.reshape(B, S // 128, TOP_K, 128)
    return x.transpose(0, 1, 3, 2).reshape(B, S, TOP_K)

  return (detile(eidx), detile(wgt), detile(pay))
